# Initial kernel scaffold; baseline (speedup 1.0000x reference)
import jax
import jax.numpy as jnp
from jax.experimental import pallas as pl


def kernel(x, edge_index, W1, b1, W2, b2):
    raise NotImplementedError("write your pallas kernel here")


# speedup vs baseline: 24.7328x; 24.7328x over previous
import functools

import jax
import jax.numpy as jnp
from jax import lax
from jax.experimental import pallas as pl
from jax.experimental.pallas import tpu as pltpu
from jax.experimental.pallas import tpu_sc as plsc

NC = 2
NS = 16
NW = NC * NS
L = 16
CHUNK = 128

_SC_PARAMS = pltpu.CompilerParams(use_tc_tiling_on_sc=False)


def _make_deg_kernel(NPD, RPTD, CPT):
    mesh = plsc.VectorSubcoreMesh(core_axis_name="c", subcore_axis_name="s")

    @functools.partial(
        pl.kernel,
        mesh=mesh,
        out_type=jax.ShapeDtypeStruct((NW * RPTD,), jnp.float32),
        scratch_types=[
            pltpu.VMEM_SHARED((NPD,), jnp.float32),
            pltpu.VMEM((CPT, CHUNK), jnp.int32),
            pltpu.VMEM((CHUNK,), jnp.float32),
            pltpu.VMEM((RPTD,), jnp.float32),
        ],
        compiler_params=_SC_PARAMS,
    )
    def deg_kernel(dst_hbm, out_hbm, deg_sp, dst_v, ones_v, stage_v):
        c = lax.axis_index("c")
        s = lax.axis_index("s")
        wid = c * NS + s

        def fill_body(i, carry):
            ones_v[pl.ds(i * L, L)] = jnp.ones((L,), jnp.float32)
            return carry

        lax.fori_loop(0, CHUNK // L, fill_body, 0)

        def zero_body(i, carry):
            stage_v[pl.ds(i * L, L)] = jnp.zeros((L,), jnp.float32)
            return carry

        lax.fori_loop(0, RPTD // L, zero_body, 0)
        pltpu.sync_copy(stage_v, deg_sp.at[pl.ds(s * RPTD, RPTD)])
        pltpu.sync_copy(dst_hbm.at[wid], dst_v)
        plsc.subcore_barrier()

        def edge_body(j, carry):
            pltpu.sync_copy(ones_v, deg_sp.at[dst_v.at[j]], add=True)
            return carry

        lax.fori_loop(0, CPT, edge_body, 0)
        plsc.subcore_barrier()
        pltpu.sync_copy(deg_sp.at[pl.ds(s * RPTD, RPTD)], stage_v)
        pltpu.sync_copy(stage_v, out_hbm.at[pl.ds(wid * RPTD, RPTD)])

    return deg_kernel


def _make_agg_kernel(NP, RPT, CPT, W):
    mesh = plsc.VectorSubcoreMesh(core_axis_name="c", subcore_axis_name="s")

    @functools.partial(
        pl.kernel,
        mesh=mesh,
        out_type=jax.ShapeDtypeStruct((NC, NP, W), jnp.float32),
        scratch_types=[
            pltpu.VMEM_SHARED((NP, W), jnp.float32),
            pltpu.VMEM((CPT, CHUNK), jnp.int32),
            pltpu.VMEM((CPT, CHUNK), jnp.int32),
            pltpu.VMEM((CHUNK, W), jnp.float32),
            pltpu.VMEM((RPT, W), jnp.float32),
            pltpu.SemaphoreType.DMA,
        ],
        compiler_params=_SC_PARAMS,
    )
    def agg_kernel(h_hbm, src_hbm, dst_hbm, out_hbm,
                   acc_sp, src_v, dst_v, rows_v, stage_v, sem):
        c = lax.axis_index("c")
        s = lax.axis_index("s")
        wid = c * NS + s

        def zero_body(t, carry):
            r = t // (W // L)
            k = t % (W // L)
            stage_v[r, pl.ds(k * L, L)] = jnp.zeros((L,), jnp.float32)
            return carry

        lax.fori_loop(0, RPT * (W // L), zero_body, 0)
        pltpu.sync_copy(stage_v, acc_sp.at[pl.ds(s * RPT, RPT)])

        pltpu.sync_copy(src_hbm.at[wid], src_v)
        pltpu.sync_copy(dst_hbm.at[wid], dst_v)
        plsc.subcore_barrier()

        def edge_body(j, carry):
            pltpu.async_copy(h_hbm.at[src_v.at[j]], rows_v, sem).wait()
            pltpu.sync_copy(rows_v, acc_sp.at[dst_v.at[j]], add=True)
            return carry

        lax.fori_loop(0, CPT, edge_body, 0)
        plsc.subcore_barrier()

        pltpu.sync_copy(acc_sp.at[pl.ds(s * RPT, RPT)], stage_v)
        pltpu.sync_copy(stage_v, out_hbm.at[c, pl.ds(s * RPT, RPT)])

    return agg_kernel


def _dinv_from_partials(degp_ref):
    deg = jnp.sum(degp_ref[...], axis=1, keepdims=True) + 1.0
    return lax.rsqrt(deg)


def _mm1_body(x_ref, w_ref, degp_ref, o_ref):
    dinv = _dinv_from_partials(degp_ref)
    h = jnp.dot(x_ref[...], w_ref[...], preferred_element_type=jnp.float32)
    o_ref[...] = h * dinv


def _mid_body(a0_ref, a1_ref, hs_ref, degp_ref, w2_ref, b1_ref, o_ref):
    dinv = _dinv_from_partials(degp_ref)
    out1 = dinv * (a0_ref[...] + a1_ref[...] + hs_ref[...]) + b1_ref[...]
    z = jnp.maximum(out1, 0.0)
    h2 = jnp.dot(z, w2_ref[...], preferred_element_type=jnp.float32)
    o_ref[...] = h2 * dinv


def _make_fin_body(C):
    def fin_body(a0_ref, a1_ref, hs2_ref, degp_ref, b2_ref, o_ref):
        dinv = _dinv_from_partials(degp_ref)
        logits = dinv * (a0_ref[...] + a1_ref[...] + hs2_ref[...]) + b2_ref[...]
        col = lax.broadcasted_iota(jnp.int32, logits.shape, 1)
        valid = col < C
        m = jnp.max(jnp.where(valid, logits, -jnp.inf), axis=1, keepdims=True)
        e = jnp.where(valid, jnp.exp(logits - m), 0.0)
        o_ref[...] = e / jnp.sum(e, axis=1, keepdims=True)

    return fin_body


@jax.jit
def kernel(x, edge_index, W1, b1, W2, b2):
    N, F = x.shape
    H = W1.shape[1]
    C = W2.shape[1]
    E = edge_index.shape[1]

    RPT = -(-(N + 1) // (NS * 8)) * 8
    NP = NS * RPT
    RPTD = -(-(N + 1) // (NS * 128)) * 128
    NPD = NS * RPTD
    CPT = -(-E // (NW * CHUNK))
    EP = NW * CHUNK * CPT
    W2L = 16

    src = edge_index[0]
    dst = edge_index[1]
    pad_idx = jnp.full((EP - E,), N, jnp.int32)
    src3 = jnp.concatenate([src, pad_idx]).reshape(NW, CPT, CHUNK)
    dst3 = jnp.concatenate([dst, pad_idx]).reshape(NW, CPT, CHUNK)

    degp = _make_deg_kernel(NPD, RPTD, CPT)(dst3)
    degp = degp.reshape(NC, NPD)
    degpT = degp[:, :N].T

    hs = pl.pallas_call(
        _mm1_body,
        out_shape=jax.ShapeDtypeStruct((N, H), jnp.float32),
    )(x, W1, degpT)

    hs_pad = jnp.pad(hs, ((0, NP - N), (0, 0)))
    accp = _make_agg_kernel(NP, RPT, CPT, H)(hs_pad, src3, dst3)
    a0 = accp[0, :N, :]
    a1 = accp[1, :N, :]

    W2p = jnp.pad(W2, ((0, 0), (0, 128 - C)))
    h2s = pl.pallas_call(
        _mid_body,
        out_shape=jax.ShapeDtypeStruct((N, 128), jnp.float32),
    )(a0, a1, hs, degpT, W2p, b1[None, :])

    hs2 = h2s[:, :W2L]
    hs2_pad = jnp.pad(hs2, ((0, NP - N), (0, 0)))
    acc2p = _make_agg_kernel(NP, RPT, CPT, W2L)(hs2_pad, src3, dst3)

    b2p = jnp.pad(b2, (0, W2L - C))[None, :]
    out = pl.pallas_call(
        _make_fin_body(C),
        out_shape=jax.ShapeDtypeStruct((N, W2L), jnp.float32),
    )(acc2p[0, :N, :], acc2p[1, :N, :], hs2, degpT, b2p)
    return out[:, :C]


# speedup vs baseline: 24.7732x; 1.0016x over previous
import functools

import jax
import jax.numpy as jnp
from jax import lax
from jax.experimental import pallas as pl
from jax.experimental.pallas import tpu as pltpu
from jax.experimental.pallas import tpu_sc as plsc

NC = 2
NS = 16
NW = NC * NS
L = 16
CHUNK = 128

_SC_PARAMS = pltpu.CompilerParams(use_tc_tiling_on_sc=False)


def _make_deg_kernel(NPD, RPTD, CPT):
    mesh = plsc.VectorSubcoreMesh(core_axis_name="c", subcore_axis_name="s")

    @functools.partial(
        pl.kernel,
        mesh=mesh,
        out_type=jax.ShapeDtypeStruct((NW * RPTD,), jnp.float32),
        scratch_types=[
            pltpu.VMEM_SHARED((NPD,), jnp.float32),
            pltpu.VMEM((CPT, CHUNK), jnp.int32),
            pltpu.VMEM((CHUNK,), jnp.float32),
            pltpu.VMEM((RPTD,), jnp.float32),
        ],
        compiler_params=_SC_PARAMS,
    )
    def deg_kernel(dst_hbm, out_hbm, deg_sp, dst_v, ones_v, stage_v):
        c = lax.axis_index("c")
        s = lax.axis_index("s")
        wid = c * NS + s

        def fill_body(i, carry):
            ones_v[pl.ds(i * L, L)] = jnp.ones((L,), jnp.float32)
            return carry

        lax.fori_loop(0, CHUNK // L, fill_body, 0)

        def zero_body(i, carry):
            stage_v[pl.ds(i * L, L)] = jnp.zeros((L,), jnp.float32)
            return carry

        lax.fori_loop(0, RPTD // L, zero_body, 0)
        pltpu.sync_copy(stage_v, deg_sp.at[pl.ds(s * RPTD, RPTD)])
        pltpu.sync_copy(dst_hbm.at[wid], dst_v)
        plsc.subcore_barrier()

        def edge_body(j, carry):
            pltpu.sync_copy(ones_v, deg_sp.at[dst_v.at[j]], add=True)
            return carry

        lax.fori_loop(0, CPT, edge_body, 0)
        plsc.subcore_barrier()
        pltpu.sync_copy(deg_sp.at[pl.ds(s * RPTD, RPTD)], stage_v)
        pltpu.sync_copy(stage_v, out_hbm.at[pl.ds(wid * RPTD, RPTD)])

    return deg_kernel


K = 4
NSETS = 2


def _make_agg_kernel(NP, RPT, CPT, W):
    mesh = plsc.VectorSubcoreMesh(core_axis_name="c", subcore_axis_name="s")
    assert CPT % (NSETS * K) == 0

    @functools.partial(
        pl.kernel,
        mesh=mesh,
        out_type=jax.ShapeDtypeStruct((NC, NP, W), jnp.float32),
        scratch_types=[
            pltpu.VMEM_SHARED((NP, W), jnp.float32),
            pltpu.VMEM((CPT, CHUNK), jnp.int32),
            pltpu.VMEM((CPT, CHUNK), jnp.int32),
            pltpu.VMEM((NSETS * K, CHUNK, W), jnp.float32),
            [pltpu.SemaphoreType.DMA] * NSETS,
            [pltpu.SemaphoreType.DMA] * NSETS,
        ],
        compiler_params=_SC_PARAMS,
    )
    def agg_kernel(h_hbm, src_hbm, dst_hbm, zero_hbm, out_hbm,
                   acc_sp, src_v, dst_v, rows_v, gsem, ssem):
        c = lax.axis_index("c")
        s = lax.axis_index("s")
        wid = c * NS + s

        pltpu.sync_copy(zero_hbm.at[pl.ds(s * RPT, RPT)],
                        acc_sp.at[pl.ds(s * RPT, RPT)])
        pltpu.sync_copy(src_hbm.at[wid], src_v)
        pltpu.sync_copy(dst_hbm.at[wid], dst_v)
        plsc.subcore_barrier()

        def start_gather(p, b, j):
            pltpu.async_copy(h_hbm.at[src_v.at[j]], rows_v.at[p * K + b],
                             gsem[p])

        def drain_gathers(p):
            for b in range(K):
                pltpu.make_async_copy(h_hbm.at[src_v.at[b]],
                                      rows_v.at[p * K + b], gsem[p]).wait()

        def start_scatter(p, b, j):
            pltpu.async_copy(rows_v.at[p * K + b], acc_sp.at[dst_v.at[j]],
                             ssem[p], add=True)

        def drain_scatters(p):
            for b in range(K):
                pltpu.make_async_copy(rows_v.at[p * K + b],
                                      acc_sp.at[dst_v.at[b]], ssem[p]).wait()

        def edge_body(h, carry):
            g0 = h * NSETS * K
            g1 = g0 + K

            @pl.when(h > 0)
            def _():
                drain_scatters(0)

            for b in range(K):
                start_gather(0, b, g0 + b)

            @pl.when(h > 0)
            def _():
                drain_scatters(1)

            for b in range(K):
                start_gather(1, b, g1 + b)
            drain_gathers(0)
            for b in range(K):
                start_scatter(0, b, g0 + b)
            drain_gathers(1)
            for b in range(K):
                start_scatter(1, b, g1 + b)
            return carry

        lax.fori_loop(0, CPT // (NSETS * K), edge_body, 0)
        drain_scatters(0)
        drain_scatters(1)
        plsc.subcore_barrier()

        pltpu.sync_copy(acc_sp.at[pl.ds(s * RPT, RPT)],
                        out_hbm.at[c, pl.ds(s * RPT, RPT)])

    return agg_kernel


def _dinv_from_partials(degp_ref):
    deg = jnp.sum(degp_ref[...], axis=1, keepdims=True) + 1.0
    return lax.rsqrt(deg)


def _mm1_body(x_ref, w_ref, degp_ref, o_ref):
    dinv = _dinv_from_partials(degp_ref)
    h = jnp.dot(x_ref[...], w_ref[...], preferred_element_type=jnp.float32)
    o_ref[...] = h * dinv


def _mid_body(a0_ref, a1_ref, hs_ref, degp_ref, w2_ref, b1_ref, o_ref):
    dinv = _dinv_from_partials(degp_ref)
    out1 = dinv * (a0_ref[...] + a1_ref[...] + hs_ref[...]) + b1_ref[...]
    z = jnp.maximum(out1, 0.0)
    h2 = jnp.dot(z, w2_ref[...], preferred_element_type=jnp.float32)
    o_ref[...] = h2 * dinv


def _make_fin_body(C):
    def fin_body(a0_ref, a1_ref, hs2_ref, degp_ref, b2_ref, o_ref):
        dinv = _dinv_from_partials(degp_ref)
        logits = dinv * (a0_ref[...] + a1_ref[...] + hs2_ref[...]) + b2_ref[...]
        col = lax.broadcasted_iota(jnp.int32, logits.shape, 1)
        valid = col < C
        m = jnp.max(jnp.where(valid, logits, -jnp.inf), axis=1, keepdims=True)
        e = jnp.where(valid, jnp.exp(logits - m), 0.0)
        o_ref[...] = e / jnp.sum(e, axis=1, keepdims=True)

    return fin_body


@jax.jit
def kernel(x, edge_index, W1, b1, W2, b2):
    N, F = x.shape
    H = W1.shape[1]
    C = W2.shape[1]
    E = edge_index.shape[1]

    RPT = -(-(N + 1) // (NS * 8)) * 8
    NP = NS * RPT
    RPTD = -(-(N + 1) // (NS * 128)) * 128
    NPD = NS * RPTD
    CPT = -(-(-(-E // (NW * CHUNK))) // (NSETS * K)) * (NSETS * K)
    EP = NW * CHUNK * CPT
    W2L = 16

    src = edge_index[0]
    dst = edge_index[1]
    pad_idx = jnp.full((EP - E,), N, jnp.int32)
    src3 = jnp.concatenate([src, pad_idx]).reshape(NW, CPT, CHUNK)
    dst3 = jnp.concatenate([dst, pad_idx]).reshape(NW, CPT, CHUNK)

    degp = _make_deg_kernel(NPD, RPTD, CPT)(dst3)
    degp = degp.reshape(NC, NPD)
    degpT = degp[:, :N].T

    hs = pl.pallas_call(
        _mm1_body,
        out_shape=jax.ShapeDtypeStruct((N, H), jnp.float32),
    )(x, W1, degpT)

    hs_pad = jnp.pad(hs, ((0, NP - N), (0, 0)))
    accp = _make_agg_kernel(NP, RPT, CPT, H)(
        hs_pad, src3, dst3, jnp.zeros((NP, H), jnp.float32))
    a0 = accp[0, :N, :]
    a1 = accp[1, :N, :]

    W2p = jnp.pad(W2, ((0, 0), (0, 128 - C)))
    h2s = pl.pallas_call(
        _mid_body,
        out_shape=jax.ShapeDtypeStruct((N, 128), jnp.float32),
    )(a0, a1, hs, degpT, W2p, b1[None, :])

    hs2 = h2s[:, :W2L]
    hs2_pad = jnp.pad(hs2, ((0, NP - N), (0, 0)))
    acc2p = _make_agg_kernel(NP, RPT, CPT, W2L)(
        hs2_pad, src3, dst3, jnp.zeros((NP, W2L), jnp.float32))

    b2p = jnp.pad(b2, (0, W2L - C))[None, :]
    out = pl.pallas_call(
        _make_fin_body(C),
        out_shape=jax.ShapeDtypeStruct((N, W2L), jnp.float32),
    )(acc2p[0, :N, :], acc2p[1, :N, :], hs2, degpT, b2p)
    return out[:, :C]


# speedup vs baseline: 42.5230x; 1.7165x over previous
import functools

import jax
import jax.numpy as jnp
from jax import lax
from jax.experimental import pallas as pl
from jax.experimental.pallas import tpu as pltpu
from jax.experimental.pallas import tpu_sc as plsc

NC = 2
NS = 16
NW = NC * NS
L = 16
CHUNK = 128

_SC_PARAMS = pltpu.CompilerParams(use_tc_tiling_on_sc=False)


def _make_deg_kernel(NPD, RPTD, CPT):
    mesh = plsc.VectorSubcoreMesh(core_axis_name="c", subcore_axis_name="s")

    @functools.partial(
        pl.kernel,
        mesh=mesh,
        out_type=jax.ShapeDtypeStruct((NW * RPTD,), jnp.float32),
        scratch_types=[
            pltpu.VMEM_SHARED((NPD,), jnp.float32),
            pltpu.VMEM((CPT, CHUNK), jnp.int32),
            pltpu.VMEM((CHUNK,), jnp.float32),
            pltpu.VMEM((RPTD,), jnp.float32),
        ],
        compiler_params=_SC_PARAMS,
    )
    def deg_kernel(dst_hbm, out_hbm, deg_sp, dst_v, ones_v, stage_v):
        c = lax.axis_index("c")
        s = lax.axis_index("s")
        wid = c * NS + s

        def fill_body(i, carry):
            ones_v[pl.ds(i * L, L)] = jnp.ones((L,), jnp.float32)
            return carry

        lax.fori_loop(0, CHUNK // L, fill_body, 0)

        def zero_body(i, carry):
            stage_v[pl.ds(i * L, L)] = jnp.zeros((L,), jnp.float32)
            return carry

        lax.fori_loop(0, RPTD // L, zero_body, 0)
        pltpu.sync_copy(stage_v, deg_sp.at[pl.ds(s * RPTD, RPTD)])
        pltpu.sync_copy(dst_hbm.at[wid], dst_v)
        plsc.subcore_barrier()

        def edge_body(j, carry):
            pltpu.sync_copy(ones_v, deg_sp.at[dst_v.at[j]], add=True)
            return carry

        lax.fori_loop(0, CPT, edge_body, 0)
        plsc.subcore_barrier()
        pltpu.sync_copy(deg_sp.at[pl.ds(s * RPTD, RPTD)], stage_v)
        pltpu.sync_copy(stage_v, out_hbm.at[pl.ds(wid * RPTD, RPTD)])

    return deg_kernel


K = 4
NSETS = 2


def _make_agg_kernel(NP, RPT, CPT, W):
    mesh = plsc.VectorSubcoreMesh(core_axis_name="c", subcore_axis_name="s")
    assert CPT % (NSETS * K) == 0

    @functools.partial(
        pl.kernel,
        mesh=mesh,
        out_type=jax.ShapeDtypeStruct((NC, NP, W), jnp.float32),
        scratch_types=[
            pltpu.VMEM_SHARED((NP, W), jnp.float32),
            pltpu.VMEM((CPT, CHUNK), jnp.int32),
            pltpu.VMEM((CPT, CHUNK), jnp.int32),
            pltpu.VMEM((NSETS * K, CHUNK, W), jnp.float32),
            [pltpu.SemaphoreType.DMA] * NSETS,
            [pltpu.SemaphoreType.DMA] * NSETS,
        ],
        compiler_params=_SC_PARAMS,
    )
    def agg_kernel(h_hbm, src_hbm, dst_hbm, zero_hbm, out_hbm,
                   acc_sp, src_v, dst_v, rows_v, gsem, ssem):
        c = lax.axis_index("c")
        s = lax.axis_index("s")
        wid = c * NS + s

        pltpu.sync_copy(zero_hbm.at[pl.ds(s * RPT, RPT)],
                        acc_sp.at[pl.ds(s * RPT, RPT)])
        pltpu.sync_copy(src_hbm.at[wid], src_v)
        pltpu.sync_copy(dst_hbm.at[wid], dst_v)
        plsc.subcore_barrier()

        def start_gather(p, b, j):
            pltpu.async_copy(h_hbm.at[src_v.at[j]], rows_v.at[p * K + b],
                             gsem[p])

        def drain_gathers(p):
            for b in range(K):
                pltpu.make_async_copy(h_hbm.at[src_v.at[b]],
                                      rows_v.at[p * K + b], gsem[p]).wait()

        def start_scatter(p, b, j):
            pltpu.async_copy(rows_v.at[p * K + b], acc_sp.at[dst_v.at[j]],
                             ssem[p], add=True)

        def drain_scatters(p):
            for b in range(K):
                pltpu.make_async_copy(rows_v.at[p * K + b],
                                      acc_sp.at[dst_v.at[b]], ssem[p]).wait()

        def edge_body(h, carry):
            g0 = h * NSETS * K
            g1 = g0 + K

            @pl.when(h > 0)
            def _():
                drain_scatters(0)

            for b in range(K):
                start_gather(0, b, g0 + b)

            @pl.when(h > 0)
            def _():
                drain_scatters(1)

            for b in range(K):
                start_gather(1, b, g1 + b)
            drain_gathers(0)
            for b in range(K):
                start_scatter(0, b, g0 + b)
            drain_gathers(1)
            for b in range(K):
                start_scatter(1, b, g1 + b)
            return carry

        lax.fori_loop(0, CPT // (NSETS * K), edge_body, 0)
        drain_scatters(0)
        drain_scatters(1)
        plsc.subcore_barrier()

        pltpu.sync_copy(acc_sp.at[pl.ds(s * RPT, RPT)],
                        out_hbm.at[c, pl.ds(s * RPT, RPT)])

    return agg_kernel


def _dinv_from_partials(degp_ref):
    deg = jnp.sum(degp_ref[...], axis=1, keepdims=True) + 1.0
    return lax.rsqrt(deg)


def _mm1_body(x_ref, w_ref, degp_ref, o_ref):
    dinv = _dinv_from_partials(degp_ref)
    h = jnp.dot(x_ref[...], w_ref[...], preferred_element_type=jnp.float32)
    o_ref[...] = h * dinv


def _mid_body(a0_ref, a1_ref, hs_ref, degp_ref, w2_ref, b1_ref, o_ref):
    dinv = _dinv_from_partials(degp_ref)
    out1 = dinv * (a0_ref[...] + a1_ref[...] + hs_ref[...]) + b1_ref[...]
    z = jnp.maximum(out1, 0.0)
    h2 = jnp.dot(z, w2_ref[...], preferred_element_type=jnp.float32)
    o_ref[...] = h2 * dinv


def _make_fin_body(C):
    def fin_body(a0_ref, a1_ref, hs2_ref, degp_ref, b2_ref, o_ref):
        dinv = _dinv_from_partials(degp_ref)
        logits = dinv * (a0_ref[...] + a1_ref[...] + hs2_ref[...]) + b2_ref[...]
        col = lax.broadcasted_iota(jnp.int32, logits.shape, 1)
        valid = col < C
        m = jnp.max(jnp.where(valid, logits, -jnp.inf), axis=1, keepdims=True)
        e = jnp.where(valid, jnp.exp(logits - m), 0.0)
        o_ref[...] = e / jnp.sum(e, axis=1, keepdims=True)

    return fin_body


@jax.jit
def kernel(x, edge_index, W1, b1, W2, b2):
    N, F = x.shape
    H = W1.shape[1]
    C = W2.shape[1]
    E = edge_index.shape[1]

    RPT = -(-(N + 1) // (NS * 8)) * 8
    NP = NS * RPT
    RPTD = -(-(N + 1) // (NS * 128)) * 128
    NPD = NS * RPTD
    CPT = -(-(-(-E // (NW * CHUNK))) // (NSETS * K)) * (NSETS * K)
    EP = NW * CHUNK * CPT
    W2L = 16

    def pad_edges(e):
        ew = -(-E // NW)
        e = jnp.concatenate(
            [e, jnp.full((NW * ew - E,), N, jnp.int32)]).reshape(NW, ew)
        padw = CPT * CHUNK - ew
        padvals = N + (jnp.arange(padw, dtype=jnp.int32) % (NP - N))
        padblk = jnp.broadcast_to(padvals, (NW, padw))
        return jnp.concatenate([e, padblk], axis=1).reshape(NW, CPT, CHUNK)

    src3 = pad_edges(edge_index[0])
    dst3 = pad_edges(edge_index[1])

    degp = _make_deg_kernel(NPD, RPTD, CPT)(dst3)
    degp = degp.reshape(NC, NPD)
    degpT = degp[:, :N].T

    hs = pl.pallas_call(
        _mm1_body,
        out_shape=jax.ShapeDtypeStruct((N, H), jnp.float32),
    )(x, W1, degpT)

    hs_pad = jnp.pad(hs, ((0, NP - N), (0, 0)))
    accp = _make_agg_kernel(NP, RPT, CPT, H)(
        hs_pad, src3, dst3, jnp.zeros((NP, H), jnp.float32))
    a0 = accp[0, :N, :]
    a1 = accp[1, :N, :]

    W2p = jnp.pad(W2, ((0, 0), (0, 128 - C)))
    h2s = pl.pallas_call(
        _mid_body,
        out_shape=jax.ShapeDtypeStruct((N, 128), jnp.float32),
    )(a0, a1, hs, degpT, W2p, b1[None, :])

    hs2 = h2s[:, :W2L]
    hs2_pad = jnp.pad(hs2, ((0, NP - N), (0, 0)))
    acc2p = _make_agg_kernel(NP, RPT, CPT, W2L)(
        hs2_pad, src3, dst3, jnp.zeros((NP, W2L), jnp.float32))

    b2p = jnp.pad(b2, (0, W2L - C))[None, :]
    out = pl.pallas_call(
        _make_fin_body(C),
        out_shape=jax.ShapeDtypeStruct((N, W2L), jnp.float32),
    )(acc2p[0, :N, :], acc2p[1, :N, :], hs2, degpT, b2p)
    return out[:, :C]
